# baseline (speedup 1.0000x reference)
import functools

import jax
import jax.numpy as jnp
from jax import lax
from jax.experimental import pallas as pl
from jax.experimental.pallas import tpu as pltpu
from jax.experimental.pallas import tpu_sc as plsc

NUM_USERS = 1024
NUM_ITEMS = 4096
TOPK = 10

_E = 1048576
_NS = 16
_EPT = _E // _NS
_CH = 2048
_NCH = _EPT // _CH
_RPP = 256
_CELLS = _RPP * NUM_ITEMS
_WPT = _CELLS // _NS


def _hist_body(edges, out, src_v, dst_v, idx_v, val_v, zero_v, flushi_v,
               flushv_v, drain_v, acc):
    c = lax.axis_index("c")
    s = lax.axis_index("s")

    def zero16(k, _):
        zero_v[pl.ds(k * 16, 16)] = jnp.zeros((16,), jnp.float32)
        return _

    lax.fori_loop(0, 4096 // 16, zero16, None)

    def flush_init(k, _):
        flushi_v[pl.ds(k * 16, 16)] = jnp.full((16,), _CELLS, jnp.int32)
        flushv_v[pl.ds(k * 16, 16)] = jnp.zeros((16,), jnp.float32)
        return _

    lax.fori_loop(0, 128 // 16, flush_init, None)

    for p in range(2):
        rowbase = c * 512 + p * _RPP
        def zslice(k, _):
            pltpu.sync_copy(zero_v, acc.at[pl.ds(s * _WPT + k * 4096, 4096)])
            return _

        lax.fori_loop(0, _WPT // 4096, zslice, None)
        plsc.subcore_barrier()

        def chunk(ch, _):
            off = s * _EPT + ch * _CH
            pltpu.sync_copy(edges.at[0, pl.ds(off, _CH)], src_v)
            pltpu.sync_copy(edges.at[1, pl.ds(off, _CH)], dst_v)

            def vec(j, _):
                sv = src_v[pl.ds(j * 16, 16)]
                dv = dst_v[pl.ds(j * 16, 16)]
                m = (sv >= rowbase) & (sv < rowbase + _RPP) & (dv >= NUM_USERS)
                idx = jnp.where(m, (sv - rowbase) * NUM_ITEMS
                                + (dv - NUM_USERS), _CELLS)
                val = jnp.where(m, 1.0, 0.0).astype(jnp.float32)
                r = j // 8
                col = (j % 8) * 16
                idx_v[r, pl.ds(col, 16)] = idx
                val_v[r, pl.ds(col, 16)] = val
                return _

            lax.fori_loop(0, _CH // 16, vec, None)

            def scat(k, _):
                pltpu.sync_copy(val_v.at[k], acc.at[idx_v.at[k]], add=True)
                return _

            lax.fori_loop(0, _CH // 128, scat, None)
            return _

        lax.fori_loop(0, _NCH, chunk, None)
        pltpu.sync_copy(flushv_v, acc.at[flushi_v], add=True)
        pltpu.sync_copy(acc.at[pl.ds(s * _WPT, 128)], drain_v)
        plsc.subcore_barrier()
        pltpu.sync_copy(
            acc.at[pl.ds(s * _WPT, _WPT)],
            out.at[pl.ds(rowbase * NUM_ITEMS + s * _WPT, _WPT)])
        plsc.subcore_barrier()


def _sc_histogram(edge_index):
    hist = functools.partial(
        pl.kernel,
        out_type=jax.ShapeDtypeStruct((NUM_USERS * NUM_ITEMS,), jnp.float32),
        mesh=plsc.VectorSubcoreMesh(core_axis_name="c", subcore_axis_name="s"),
        scratch_types=[
            pltpu.VMEM((_CH,), jnp.int32),
            pltpu.VMEM((_CH,), jnp.int32),
            pltpu.VMEM((16, 128), jnp.int32),
            pltpu.VMEM((16, 128), jnp.float32),
            pltpu.VMEM((4096,), jnp.float32),
            pltpu.VMEM((128,), jnp.int32),
            pltpu.VMEM((128,), jnp.float32),
            pltpu.VMEM((128,), jnp.float32),
            pltpu.VMEM_SHARED((_CELLS + 128,), jnp.float32),
        ],
    )(_hist_body)
    return hist(edge_index).reshape(NUM_USERS, NUM_ITEMS)


def _sums_body(cnt_ref, cu_ref, ci_ref):
    cnt = cnt_ref[...]
    cu_ref[...] = jnp.sum(cnt, axis=1)
    ci_ref[...] = jnp.sum(cnt, axis=0)


def _dense_body(cnt_ref, du_ref, di_ref, wi_ref, s_ref):
    cnt = cnt_ref[...]
    su = jnp.sqrt(du_ref[...])
    sdi = jnp.sqrt(di_ref[...])
    x = (su[:, None] * sdi[None, :]) * wi_ref[...][None, :]
    a = jnp.zeros_like(cnt)
    for t in range(8):
        a = a + jnp.where(cnt > t, x, 0.0)
    a = a + jnp.maximum(cnt - 8.0, 0.0) * x
    c0 = jax.lax.dot_general(a, a, (((1,), (1,)), ((), ())),
                             preferred_element_type=jnp.float32)
    c = (c0 * su[:, None]) * su[None, :]
    iota = jax.lax.broadcasted_iota(jnp.int32, (NUM_USERS, NUM_USERS), 1)
    s0 = jnp.zeros((NUM_USERS, NUM_USERS), dtype=jnp.float32)
    for _ in range(TOPK):
        m = jnp.max(c, axis=1, keepdims=True)
        first = jnp.min(jnp.where(c == m, iota, NUM_USERS), axis=1,
                        keepdims=True)
        onehot = iota == first
        s0 = s0 + jnp.where(onehot & (m > 0), m * 0.5, 0.0)
        c = jnp.where(onehot, -jnp.inf, c)
    s_ref[...] = s0 + s0.T


def kernel(edge_index, user_idx, num_nodes, fraud_label_i, W1, b1, W2, b2):
    cnt = _sc_histogram(edge_index)
    cnt_u, cnt_i = pl.pallas_call(
        _sums_body,
        out_shape=(
            jax.ShapeDtypeStruct((NUM_USERS,), jnp.float32),
            jax.ShapeDtypeStruct((NUM_ITEMS,), jnp.float32),
        ),
    )(cnt)
    d_u = jnp.log1p(cnt_u)
    d_i = jnp.log1p(cnt_i)
    x_i = jnp.stack([d_i, fraud_label_i], axis=-1)
    h = jax.nn.relu(x_i @ W1.T + b1)
    w_i = jax.nn.sigmoid(h @ W2.T + b2).squeeze(-1)
    s = pl.pallas_call(
        _dense_body,
        out_shape=jax.ShapeDtypeStruct((NUM_USERS, NUM_USERS), jnp.float32),
    )(cnt, d_u, d_i, w_i)
    return s, d_u
